# baseline (speedup 1.0000x reference)
import dataclasses
import functools

import jax
import jax.numpy as jnp
from jax import lax
from jax.experimental import pallas as pl
from jax.experimental.pallas import tpu as pltpu
from jax.experimental.pallas import tpu_sc as plsc

N = 10000
E = 160000
IN_FEAT = 256
OUT_FEAT = 128
NUM_HEADS = 5
NUM_RELS = 20

HP = 16
EP = 163840
NTILES = 32
EDGES_PER_TILE = EP // NTILES
BLK = 64
NBLK = EDGES_PER_TILE // BLK
ACC_ROWS = 10240
ZERO_ROWS = ACC_ROWS // 16


def _proj_body(feat_ref, wfc_ref, wself_ref, w1_ref, w2_ref,
               z_ref, selfz_ref, s1_ref, s2_ref):
    f = feat_ref[...]
    dn = (((1,), (1,)), ((), ()))
    z = lax.dot_general(f, wfc_ref[...], dn,
                        precision=lax.Precision.HIGHEST,
                        preferred_element_type=jnp.float32)
    z_ref[...] = z
    selfz_ref[...] = lax.dot_general(f, wself_ref[...], dn,
                                     precision=lax.Precision.HIGHEST,
                                     preferred_element_type=jnp.float32)
    s1_ref[...] = jnp.dot(z, w1_ref[...], precision=lax.Precision.HIGHEST,
                          preferred_element_type=jnp.float32)
    s2_ref[...] = jnp.dot(z, w2_ref[...], precision=lax.Precision.HIGHEST,
                          preferred_element_type=jnp.float32)


def _tc_proj(feat, wfc, wself, w1m, w2m):
    BN = 1000
    sw = NUM_RELS * HP
    return pl.pallas_call(
        _proj_body,
        grid=(N // BN,),
        in_specs=[
            pl.BlockSpec((BN, IN_FEAT), lambda i: (i, 0)),
            pl.BlockSpec((OUT_FEAT, IN_FEAT), lambda i: (0, 0)),
            pl.BlockSpec((NUM_HEADS * OUT_FEAT, IN_FEAT), lambda i: (0, 0)),
            pl.BlockSpec((OUT_FEAT, sw), lambda i: (0, 0)),
            pl.BlockSpec((OUT_FEAT, sw), lambda i: (0, 0)),
        ],
        out_specs=[
            pl.BlockSpec((BN, OUT_FEAT), lambda i: (i, 0)),
            pl.BlockSpec((BN, NUM_HEADS * OUT_FEAT), lambda i: (i, 0)),
            pl.BlockSpec((BN, sw), lambda i: (i, 0)),
            pl.BlockSpec((BN, sw), lambda i: (i, 0)),
        ],
        out_shape=[
            jax.ShapeDtypeStruct((N, OUT_FEAT), jnp.float32),
            jax.ShapeDtypeStruct((N, NUM_HEADS * OUT_FEAT), jnp.float32),
            jax.ShapeDtypeStruct((N, sw), jnp.float32),
            jax.ShapeDtypeStruct((N, sw), jnp.float32),
        ],
    )(feat, wfc, wself, w1m, w2m)


def _combine_body(p_ref, selfz_ref, out_ref):
    for h in range(NUM_HEADS):
        sl = slice(h * OUT_FEAT, (h + 1) * OUT_FEAT)
        out_ref[:, sl] = p_ref[0, h] + p_ref[1, h] + selfz_ref[:, sl]


def _tc_combine(partial, self_z):
    BN = 1000
    return pl.pallas_call(
        _combine_body,
        grid=(N // BN,),
        in_specs=[
            pl.BlockSpec((2, NUM_HEADS, BN, OUT_FEAT), lambda i: (0, 0, i, 0)),
            pl.BlockSpec((BN, NUM_HEADS * OUT_FEAT), lambda i: (i, 0)),
        ],
        out_specs=pl.BlockSpec((BN, NUM_HEADS * OUT_FEAT), lambda i: (i, 0)),
        out_shape=jax.ShapeDtypeStruct((N, NUM_HEADS * OUT_FEAT), jnp.float32),
    )(partial, self_z)


def _sc_edge_kernel(z_hbm, s12_hbm, src_hbm, dst_hbm, et_hbm, out_hbm,
                    w0_all, dst_all,
                    s12A, zA, msgA, sixA, srcbA, dstbA,
                    s12B, zB, msgB, sixB, srcbB, dstbB,
                    att_v, acc,
                    semGA, semGB, semSA, semSB):
    c = lax.axis_index("c")
    s = lax.axis_index("s")
    tid = c * 16 + s
    base_edge = tid * EDGES_PER_TILE

    pltpu.sync_copy(src_hbm.at[pl.ds(base_edge, EDGES_PER_TILE)], w0_all)
    pltpu.sync_copy(et_hbm.at[pl.ds(base_edge, EDGES_PER_TILE)], dst_all)

    @plsc.parallel_loop(0, EDGES_PER_TILE, step=16, unroll=4)
    def _pk(e0):
        sl = pl.ds(e0, 16)
        w0_all[sl] = w0_all[sl] * NUM_RELS + dst_all[sl]

    pltpu.sync_copy(dst_hbm.at[pl.ds(base_edge, EDGES_PER_TILE)], dst_all)

    def start_gathers(b, s12b, zb, six, srcb, sem):
        @plsc.parallel_loop(0, BLK, step=16, unroll=2)
        def _ux(k0):
            sl = pl.ds(k0, 16)
            w0 = w0_all[pl.ds(b * BLK + k0, 16)]
            dv = dst_all[pl.ds(b * BLK + k0, 16)]
            sv = w0 // NUM_RELS
            srcb[sl] = sv
            six[sl] = w0
            six[pl.ds(BLK + k0, 16)] = (N * NUM_RELS + dv * NUM_RELS
                                        + (w0 - sv * NUM_RELS))

        pltpu.async_copy(s12_hbm.at[six], s12b, sem)
        pltpu.async_copy(z_hbm.at[srcb], zb, sem)

    def wait_gathers(b, s12b, zb, six, srcb, sem):
        pltpu.make_async_copy(s12_hbm.at[six], s12b, sem).wait()
        pltpu.make_async_copy(z_hbm.at[srcb], zb, sem).wait()

    def wait_scatter(msgb, dstb, sem):
        pltpu.make_async_copy(msgb, acc.at[dstb], sem).wait()

    def compute_block(b, s12b, zb, msgb, dstb, h):
        @plsc.parallel_loop(0, BLK, step=16, unroll=2)
        def _att(e0):
            sl = pl.ds(e0, 16)
            dstb[sl] = dst_all[pl.ds(b * BLK + e0, 16)]
            ids = lax.iota(jnp.int32, 16) + e0
            hh = jnp.zeros((16,), jnp.int32) + h
            a = (plsc.load_gather(s12b, [ids, hh])
                 + plsc.load_gather(s12b, [ids + BLK, hh]))
            att_v[sl] = jnp.maximum(a, 0.0) + 0.01 * jnp.minimum(a, 0.0)

        @plsc.parallel_loop(0, BLK, unroll=4)
        def _msg(e):
            ai = plsc.load_gather(att_v, [jnp.zeros((16,), jnp.int32) + e])
            for cc in range(OUT_FEAT // 16):
                sl = pl.ds(cc * 16, 16)
                msgb[e, sl] = ai * zb[e, sl]

    @pl.loop(0, NUM_HEADS)
    def _head(h):
        zeros16 = jnp.zeros((16,), jnp.float32)

        @plsc.parallel_loop(0, BLK, unroll=4)
        def _zm(r):
            for cc in range(OUT_FEAT // 16):
                msgA[r, pl.ds(cc * 16, 16)] = zeros16

        @pl.loop(0, ZERO_ROWS // BLK)
        def _za(i):
            pltpu.async_copy(msgA, acc.at[pl.ds(s * ZERO_ROWS + i * BLK, BLK)],
                             semSA)

        @pl.loop(0, ZERO_ROWS // BLK)
        def _zw(i):
            pltpu.make_async_copy(
                msgA, acc.at[pl.ds(s * ZERO_ROWS + i * BLK, BLK)], semSA).wait()

        plsc.subcore_barrier()

        start_gathers(0, s12A, zA, sixA, srcbA, semGA)
        start_gathers(1, s12B, zB, sixB, srcbB, semGB)

        @pl.loop(0, NBLK // 2)
        def _blk(p):
            b0 = 2 * p

            wait_gathers(b0, s12A, zA, sixA, srcbA, semGA)

            @pl.when(p > 0)
            def _wsa():
                wait_scatter(msgA, dstbA, semSA)

            compute_block(b0, s12A, zA, msgA, dstbA, h)
            pltpu.async_copy(msgA, acc.at[dstbA], semSA, add=True)

            @pl.when(p < NBLK // 2 - 1)
            def _nga():
                start_gathers(b0 + 2, s12A, zA, sixA, srcbA, semGA)

            wait_gathers(b0 + 1, s12B, zB, sixB, srcbB, semGB)

            @pl.when(p > 0)
            def _wsb():
                wait_scatter(msgB, dstbB, semSB)

            compute_block(b0 + 1, s12B, zB, msgB, dstbB, h)
            pltpu.async_copy(msgB, acc.at[dstbB], semSB, add=True)

            @pl.when(p < NBLK // 2 - 1)
            def _ngb():
                start_gathers(b0 + 3, s12B, zB, sixB, srcbB, semGB)

        wait_scatter(msgA, dstbA, semSA)
        wait_scatter(msgB, dstbB, semSB)
        plsc.subcore_barrier()
        pltpu.sync_copy(acc.at[pl.ds(s * ZERO_ROWS, ZERO_ROWS)],
                        out_hbm.at[c, h, pl.ds(s * ZERO_ROWS, ZERO_ROWS)])
        plsc.subcore_barrier()


def _sc_edge(z, s12f, src_p, dst_p, et_p):
    mesh = plsc.VectorSubcoreMesh(core_axis_name="c", subcore_axis_name="s")
    cp = pltpu.CompilerParams(use_tc_tiling_on_sc=False)
    if "needs_layout_passes" in pltpu.CompilerParams.__dataclass_fields__:
        cp = dataclasses.replace(cp, needs_layout_passes=False)
    run = pl.kernel(
        _sc_edge_kernel,
        out_type=jax.ShapeDtypeStruct((2, NUM_HEADS, ACC_ROWS, OUT_FEAT),
                                      jnp.float32),
        mesh=mesh,
        scratch_types=[
            pltpu.VMEM((EDGES_PER_TILE,), jnp.int32),
            pltpu.VMEM((EDGES_PER_TILE,), jnp.int32),
            pltpu.VMEM((2 * BLK, HP), jnp.float32),
            pltpu.VMEM((BLK, OUT_FEAT), jnp.float32),
            pltpu.VMEM((BLK, OUT_FEAT), jnp.float32),
            pltpu.VMEM((2 * BLK,), jnp.int32),
            pltpu.VMEM((BLK,), jnp.int32),
            pltpu.VMEM((BLK,), jnp.int32),
            pltpu.VMEM((2 * BLK, HP), jnp.float32),
            pltpu.VMEM((BLK, OUT_FEAT), jnp.float32),
            pltpu.VMEM((BLK, OUT_FEAT), jnp.float32),
            pltpu.VMEM((2 * BLK,), jnp.int32),
            pltpu.VMEM((BLK,), jnp.int32),
            pltpu.VMEM((BLK,), jnp.int32),
            pltpu.VMEM((BLK,), jnp.float32),
            pltpu.VMEM_SHARED((ACC_ROWS, OUT_FEAT), jnp.float32),
            pltpu.SemaphoreType.DMA,
            pltpu.SemaphoreType.DMA,
            pltpu.SemaphoreType.DMA,
            pltpu.SemaphoreType.DMA,
        ],
        compiler_params=cp,
    )
    return run(z, s12f, src_p, dst_p, et_p)


def kernel(feat, edge_index, edge_type, W_fc, W_self, attn_w):
    src = edge_index[0]
    dst = edge_index[1]
    pad = EP - E
    src_p = jnp.concatenate([src, jnp.zeros((pad,), jnp.int32)])
    dst_p = jnp.concatenate([dst, jnp.full((pad,), N, jnp.int32)])
    et_p = jnp.concatenate([edge_type, jnp.zeros((pad,), jnp.int32)])

    w1 = jnp.pad(attn_w[:, :OUT_FEAT, :], ((0, 0), (0, 0), (0, HP - NUM_HEADS)))
    w2 = jnp.pad(attn_w[:, OUT_FEAT:, :], ((0, 0), (0, 0), (0, HP - NUM_HEADS)))
    w1m = w1.transpose(1, 0, 2).reshape(OUT_FEAT, NUM_RELS * HP)
    w2m = w2.transpose(1, 0, 2).reshape(OUT_FEAT, NUM_RELS * HP)

    z, self_z, s1, s2 = _tc_proj(feat, W_fc, W_self, w1m, w2m)
    s12f = jnp.concatenate([s1.reshape(N * NUM_RELS, HP),
                            s2.reshape(N * NUM_RELS, HP),
                            jnp.zeros((NUM_RELS + 4, HP), jnp.float32)])

    partial = _sc_edge(z, s12f, src_p, dst_p, et_p)
    return _tc_combine(partial, self_z)


# speedup vs baseline: 1.6092x; 1.6092x over previous
import dataclasses
import functools

import jax
import jax.numpy as jnp
from jax import lax
from jax.experimental import pallas as pl
from jax.experimental.pallas import tpu as pltpu
from jax.experimental.pallas import tpu_sc as plsc

N = 10000
E = 160000
IN_FEAT = 256
OUT_FEAT = 128
NUM_HEADS = 5
NUM_RELS = 20

HP = 16
EP = 163840
NTILES = 32
EDGES_PER_TILE = EP // NTILES
BLK = 64
NBLK = EDGES_PER_TILE // BLK
ACC_ROWS = 10240
ZERO_ROWS = ACC_ROWS // 16


def _proj_body(feat_ref, wfc_ref, wself_ref, w1_ref, w2_ref,
               z_ref, selfz_ref, s1_ref, s2_ref):
    f = feat_ref[...]
    dn = (((1,), (1,)), ((), ()))
    z = lax.dot_general(f, wfc_ref[...], dn,
                        precision=lax.Precision.HIGHEST,
                        preferred_element_type=jnp.float32)
    z_ref[...] = z
    selfz_ref[...] = lax.dot_general(f, wself_ref[...], dn,
                                     precision=lax.Precision.HIGHEST,
                                     preferred_element_type=jnp.float32)
    s1_ref[...] = jnp.dot(z, w1_ref[...], precision=lax.Precision.HIGHEST,
                          preferred_element_type=jnp.float32)
    s2_ref[...] = jnp.dot(z, w2_ref[...], precision=lax.Precision.HIGHEST,
                          preferred_element_type=jnp.float32)


def _tc_proj(feat, wfc, wself, w1m, w2m):
    BN = 1000
    sw = NUM_RELS * HP
    return pl.pallas_call(
        _proj_body,
        grid=(N // BN,),
        in_specs=[
            pl.BlockSpec((BN, IN_FEAT), lambda i: (i, 0)),
            pl.BlockSpec((OUT_FEAT, IN_FEAT), lambda i: (0, 0)),
            pl.BlockSpec((NUM_HEADS * OUT_FEAT, IN_FEAT), lambda i: (0, 0)),
            pl.BlockSpec((OUT_FEAT, sw), lambda i: (0, 0)),
            pl.BlockSpec((OUT_FEAT, sw), lambda i: (0, 0)),
        ],
        out_specs=[
            pl.BlockSpec((BN, OUT_FEAT), lambda i: (i, 0)),
            pl.BlockSpec((BN, NUM_HEADS * OUT_FEAT), lambda i: (i, 0)),
            pl.BlockSpec((BN, sw), lambda i: (i, 0)),
            pl.BlockSpec((BN, sw), lambda i: (i, 0)),
        ],
        out_shape=[
            jax.ShapeDtypeStruct((N, OUT_FEAT), jnp.float32),
            jax.ShapeDtypeStruct((N, NUM_HEADS * OUT_FEAT), jnp.float32),
            jax.ShapeDtypeStruct((N, sw), jnp.float32),
            jax.ShapeDtypeStruct((N, sw), jnp.float32),
        ],
    )(feat, wfc, wself, w1m, w2m)


def _combine_body(p_ref, selfz_ref, out_ref):
    for h in range(NUM_HEADS):
        sl = slice(h * OUT_FEAT, (h + 1) * OUT_FEAT)
        out_ref[:, sl] = p_ref[0, h] + p_ref[1, h] + selfz_ref[:, sl]


def _tc_combine(partial, self_z):
    BN = 1000
    return pl.pallas_call(
        _combine_body,
        grid=(N // BN,),
        in_specs=[
            pl.BlockSpec((2, NUM_HEADS, BN, OUT_FEAT), lambda i: (0, 0, i, 0)),
            pl.BlockSpec((BN, NUM_HEADS * OUT_FEAT), lambda i: (i, 0)),
        ],
        out_specs=pl.BlockSpec((BN, NUM_HEADS * OUT_FEAT), lambda i: (i, 0)),
        out_shape=jax.ShapeDtypeStruct((N, NUM_HEADS * OUT_FEAT), jnp.float32),
    )(partial, self_z)


def _sc_edge_kernel(z_hbm, s1_hbm, s2_hbm, src_hbm, dst_hbm, et_hbm, out_hbm,
                    w0_all, dst_all,
                    s1A, s2A, zA, msgA, srcbA, i2bA, dstbA,
                    s1B, s2B, zB, msgB, srcbB, i2bB, dstbB,
                    att_v, acc,
                    semGA, semGB, semSA, semSB):
    c = lax.axis_index("c")
    s = lax.axis_index("s")
    tid = c * 16 + s
    base_edge = tid * EDGES_PER_TILE

    pltpu.sync_copy(src_hbm.at[pl.ds(base_edge, EDGES_PER_TILE)], w0_all)
    pltpu.sync_copy(et_hbm.at[pl.ds(base_edge, EDGES_PER_TILE)], dst_all)

    @plsc.parallel_loop(0, EDGES_PER_TILE, step=16, unroll=4)
    def _pk(e0):
        sl = pl.ds(e0, 16)
        w0_all[sl] = w0_all[sl] * NUM_RELS + dst_all[sl]

    pltpu.sync_copy(dst_hbm.at[pl.ds(base_edge, EDGES_PER_TILE)], dst_all)

    def start_gathers(b, s1b, s2b, zb, srcb, i2b, sem):
        @plsc.parallel_loop(0, BLK, step=16, unroll=2)
        def _ux(k0):
            sl = pl.ds(k0, 16)
            w0 = w0_all[pl.ds(b * BLK + k0, 16)]
            dv = dst_all[pl.ds(b * BLK + k0, 16)]
            sv = w0 // NUM_RELS
            srcb[sl] = sv
            i2b[sl] = dv * NUM_RELS + (w0 - sv * NUM_RELS)

        sl = pl.ds(b * BLK, BLK)
        pltpu.async_copy(s1_hbm.at[w0_all.at[sl]], s1b, sem)
        pltpu.async_copy(s2_hbm.at[i2b], s2b, sem)
        pltpu.async_copy(z_hbm.at[srcb], zb, sem)

    def wait_gathers(b, s1b, s2b, zb, srcb, i2b, sem):
        sl = pl.ds(b * BLK, BLK)
        pltpu.make_async_copy(s1_hbm.at[w0_all.at[sl]], s1b, sem).wait()
        pltpu.make_async_copy(s2_hbm.at[i2b], s2b, sem).wait()
        pltpu.make_async_copy(z_hbm.at[srcb], zb, sem).wait()

    def wait_scatter(msgb, dstb, sem):
        pltpu.make_async_copy(msgb, acc.at[dstb], sem).wait()

    def compute_block(b, s1b, s2b, zb, msgb, dstb, h):
        @plsc.parallel_loop(0, BLK, step=16, unroll=2)
        def _att(e0):
            sl = pl.ds(e0, 16)
            dstb[sl] = dst_all[pl.ds(b * BLK + e0, 16)]
            ids = lax.iota(jnp.int32, 16) + e0
            hh = jnp.zeros((16,), jnp.int32) + h
            a = (plsc.load_gather(s1b, [ids, hh])
                 + plsc.load_gather(s2b, [ids, hh]))
            att_v[sl] = jnp.maximum(a, 0.0) + 0.01 * jnp.minimum(a, 0.0)

        @plsc.parallel_loop(0, BLK, unroll=4)
        def _msg(e):
            ai = plsc.load_gather(att_v, [jnp.zeros((16,), jnp.int32) + e])
            for cc in range(OUT_FEAT // 16):
                sl = pl.ds(cc * 16, 16)
                msgb[e, sl] = ai * zb[e, sl]

    @pl.loop(0, NUM_HEADS)
    def _head(h):
        zeros16 = jnp.zeros((16,), jnp.float32)

        @plsc.parallel_loop(0, BLK, unroll=4)
        def _zm(r):
            for cc in range(OUT_FEAT // 16):
                msgA[r, pl.ds(cc * 16, 16)] = zeros16

        @pl.loop(0, ZERO_ROWS // BLK)
        def _za(i):
            pltpu.async_copy(msgA, acc.at[pl.ds(s * ZERO_ROWS + i * BLK, BLK)],
                             semSA)

        @pl.loop(0, ZERO_ROWS // BLK)
        def _zw(i):
            pltpu.make_async_copy(
                msgA, acc.at[pl.ds(s * ZERO_ROWS + i * BLK, BLK)], semSA).wait()

        plsc.subcore_barrier()

        start_gathers(0, s1A, s2A, zA, srcbA, i2bA, semGA)
        start_gathers(1, s1B, s2B, zB, srcbB, i2bB, semGB)

        @pl.loop(0, NBLK // 2)
        def _blk(p):
            b0 = 2 * p

            wait_gathers(b0, s1A, s2A, zA, srcbA, i2bA, semGA)

            @pl.when(p > 0)
            def _wsa():
                wait_scatter(msgA, dstbA, semSA)

            compute_block(b0, s1A, s2A, zA, msgA, dstbA, h)
            pltpu.async_copy(msgA, acc.at[dstbA], semSA, add=True)

            @pl.when(p < NBLK // 2 - 1)
            def _nga():
                start_gathers(b0 + 2, s1A, s2A, zA, srcbA, i2bA, semGA)

            wait_gathers(b0 + 1, s1B, s2B, zB, srcbB, i2bB, semGB)

            @pl.when(p > 0)
            def _wsb():
                wait_scatter(msgB, dstbB, semSB)

            compute_block(b0 + 1, s1B, s2B, zB, msgB, dstbB, h)
            pltpu.async_copy(msgB, acc.at[dstbB], semSB, add=True)

            @pl.when(p < NBLK // 2 - 1)
            def _ngb():
                start_gathers(b0 + 3, s1B, s2B, zB, srcbB, i2bB, semGB)

        wait_scatter(msgA, dstbA, semSA)
        wait_scatter(msgB, dstbB, semSB)
        plsc.subcore_barrier()
        pltpu.sync_copy(acc.at[pl.ds(s * ZERO_ROWS, ZERO_ROWS)],
                        out_hbm.at[c, h, pl.ds(s * ZERO_ROWS, ZERO_ROWS)])
        plsc.subcore_barrier()


def _sc_edge(z, s1f, s2f, src_p, dst_p, et_p):
    mesh = plsc.VectorSubcoreMesh(core_axis_name="c", subcore_axis_name="s")
    cp = pltpu.CompilerParams(use_tc_tiling_on_sc=False)
    if "needs_layout_passes" in pltpu.CompilerParams.__dataclass_fields__:
        cp = dataclasses.replace(cp, needs_layout_passes=False)
    run = pl.kernel(
        _sc_edge_kernel,
        out_type=jax.ShapeDtypeStruct((2, NUM_HEADS, ACC_ROWS, OUT_FEAT),
                                      jnp.float32),
        mesh=mesh,
        scratch_types=[
            pltpu.VMEM((EDGES_PER_TILE,), jnp.int32),
            pltpu.VMEM((EDGES_PER_TILE,), jnp.int32),
            pltpu.VMEM((BLK, HP), jnp.float32),
            pltpu.VMEM((BLK, HP), jnp.float32),
            pltpu.VMEM((BLK, OUT_FEAT), jnp.float32),
            pltpu.VMEM((BLK, OUT_FEAT), jnp.float32),
            pltpu.VMEM((BLK,), jnp.int32),
            pltpu.VMEM((BLK,), jnp.int32),
            pltpu.VMEM((BLK,), jnp.int32),
            pltpu.VMEM((BLK, HP), jnp.float32),
            pltpu.VMEM((BLK, HP), jnp.float32),
            pltpu.VMEM((BLK, OUT_FEAT), jnp.float32),
            pltpu.VMEM((BLK, OUT_FEAT), jnp.float32),
            pltpu.VMEM((BLK,), jnp.int32),
            pltpu.VMEM((BLK,), jnp.int32),
            pltpu.VMEM((BLK,), jnp.int32),
            pltpu.VMEM((BLK,), jnp.float32),
            pltpu.VMEM_SHARED((ACC_ROWS, OUT_FEAT), jnp.float32),
            pltpu.SemaphoreType.DMA,
            pltpu.SemaphoreType.DMA,
            pltpu.SemaphoreType.DMA,
            pltpu.SemaphoreType.DMA,
        ],
        compiler_params=cp,
    )
    return run(z, s1f, s2f, src_p, dst_p, et_p)


def kernel(feat, edge_index, edge_type, W_fc, W_self, attn_w):
    src = edge_index[0]
    dst = edge_index[1]
    pad = EP - E
    src_p = jnp.concatenate([src, jnp.zeros((pad,), jnp.int32)])
    dst_p = jnp.concatenate([dst, jnp.full((pad,), N, jnp.int32)])
    et_p = jnp.concatenate([edge_type, jnp.zeros((pad,), jnp.int32)])

    w1 = jnp.pad(attn_w[:, :OUT_FEAT, :], ((0, 0), (0, 0), (0, HP - NUM_HEADS)))
    w2 = jnp.pad(attn_w[:, OUT_FEAT:, :], ((0, 0), (0, 0), (0, HP - NUM_HEADS)))
    w1m = w1.transpose(1, 0, 2).reshape(OUT_FEAT, NUM_RELS * HP)
    w2m = w2.transpose(1, 0, 2).reshape(OUT_FEAT, NUM_RELS * HP)

    z, self_z, s1, s2 = _tc_proj(feat, W_fc, W_self, w1m, w2m)
    s1f = s1.reshape(N * NUM_RELS, HP)
    s2f = s2.reshape(N * NUM_RELS, HP)

    partial = _sc_edge(z, s1f, s2f, src_p, dst_p, et_p)
    return _tc_combine(partial, self_z)
